# Initial kernel scaffold; baseline (speedup 1.0000x reference)
import functools

import jax
import jax.numpy as jnp
from jax import lax
from jax.experimental import pallas as pl
from jax.experimental.pallas import tpu as pltpu
from jax.experimental.pallas import tpu_sc as plsc

NC = 2
NS = 16
NW = NC * NS


def _make_sc_gather(n_idx, chunk, want_tables, want_edges, kk, d_node, d_edge):
  nch = n_idx // (NW * chunk)
  assert nch * NW * chunk == n_idx
  mesh = plsc.VectorSubcoreMesh(core_axis_name="c", subcore_axis_name="s")

  out_type = []
  if want_tables:
    out_type += [jax.ShapeDtypeStruct((n_idx, kk), jnp.int32),
                 jax.ShapeDtypeStruct((n_idx, kk), jnp.int32),
                 jax.ShapeDtypeStruct((n_idx, kk), jnp.float32)]
  out_type.append(jax.ShapeDtypeStruct((n_idx, d_node), jnp.float32))
  if want_edges:
    out_type.append(jax.ShapeDtypeStruct((n_idx, d_edge), jnp.float32))

  scratch = [pltpu.VMEM((nch, chunk), jnp.int32),
             pltpu.VMEM((chunk,), jnp.int32),
             pltpu.VMEM((chunk, d_node), jnp.float32),
             pltpu.VMEM((chunk, d_node), jnp.float32),
             pltpu.SemaphoreType.DMA, pltpu.SemaphoreType.DMA]
  if want_tables:
    scratch += [pltpu.VMEM((chunk, kk), jnp.int32),
                pltpu.VMEM((chunk, kk), jnp.int32),
                pltpu.VMEM((chunk, kk), jnp.float32),
                pltpu.SemaphoreType.DMA, pltpu.SemaphoreType.DMA,
                pltpu.SemaphoreType.DMA]
  if want_edges:
    scratch += [pltpu.VMEM((nch, chunk), jnp.int32),
                pltpu.VMEM((chunk, d_edge), jnp.float32),
                pltpu.SemaphoreType.DMA]

  def body(*refs):
    it = iter(refs)
    if want_tables:
      nt_h, et_h, tt_h = next(it), next(it), next(it)
    nf_h, mem_h = next(it), next(it)
    if want_edges:
      ef_h = next(it)
    nidx_h = next(it)
    if want_edges:
      eidx_h = next(it)
    if want_tables:
      n_o, e_o, t_o = next(it), next(it), next(it)
    f_o = next(it)
    if want_edges:
      eo = next(it)
    nidx_v, ident_v, bf, bm = next(it), next(it), next(it), next(it)
    s0, s1 = next(it), next(it)
    if want_tables:
      bn, be, bt = next(it), next(it), next(it)
      s2, s3, s4 = next(it), next(it), next(it)
    if want_edges:
      eidx_v, bef = next(it), next(it)
      s5 = next(it)

    wid = lax.axis_index("s") * NC + lax.axis_index("c")

    for i in range(chunk // 16):
      ident_v[pl.ds(i * 16, 16)] = lax.iota(jnp.int32, 16) + i * 16

    pltpu.sync_copy(nidx_h.at[pl.ds(wid * nch, nch)], nidx_v)
    if want_edges:
      pltpu.sync_copy(eidx_h.at[pl.ds(wid * nch, nch)], eidx_v)

    for i in range(nch):
      base = (wid * nch + i) * chunk
      idx = nidx_v.at[i]
      cf = pltpu.async_copy(nf_h.at[idx], bf, s0)
      cm = pltpu.async_copy(mem_h.at[idx], bm, s1)
      if want_tables:
        cn = pltpu.async_copy(nt_h.at[idx], bn, s2)
        ce = pltpu.async_copy(et_h.at[idx], be, s3)
        ct = pltpu.async_copy(tt_h.at[idx], bt, s4)
      if want_edges:
        cef = pltpu.async_copy(ef_h.at[eidx_v.at[i]], bef, s5)
      cf.wait()
      cm.wait()
      pltpu.sync_copy(bm, bf.at[ident_v], add=True)
      pltpu.sync_copy(bf, f_o.at[pl.ds(base, chunk)])
      if want_tables:
        cn.wait()
        pltpu.sync_copy(bn, n_o.at[pl.ds(base, chunk)])
        ce.wait()
        pltpu.sync_copy(be, e_o.at[pl.ds(base, chunk)])
        ct.wait()
        pltpu.sync_copy(bt, t_o.at[pl.ds(base, chunk)])
      if want_edges:
        cef.wait()
        pltpu.sync_copy(bef, eo.at[pl.ds(base, chunk)])

  return pl.kernel(body, out_type=tuple(out_type), mesh=mesh,
                   scratch_types=tuple(scratch))


def _agg_big_body(fn_ref, t_ref, n_ref, e_ref, ts_ref, fs_ref, w1_ref, b1_ref,
                  w2_ref, b2_ref, tw_ref, tb_ref, out_ref, *, kk, d_node):
  j = pl.program_id(1)
  delta = ts_ref[...] - t_ref[0]
  tt = jnp.cos(delta * tw_ref[...] + tb_ref[...])
  f = fn_ref[0]
  e = e_ref[0]
  pre = (jnp.dot(f, w1_ref[0:d_node], preferred_element_type=jnp.float32)
         + jnp.dot(tt, w1_ref[d_node:2 * d_node],
                   preferred_element_type=jnp.float32)
         + jnp.dot(e, w1_ref[2 * d_node:], preferred_element_type=jnp.float32)
         + b1_ref[...])
  m = (n_ref[0] != 0).astype(jnp.float32)
  h = jnp.maximum(pre, 0.0) * m

  @pl.when(j == 0)
  def _():
    out_ref[...] = h

  @pl.when(j > 0)
  def _():
    out_ref[...] += h

  @pl.when(j == kk - 1)
  def _():
    acc = out_ref[...]
    cterm = (jnp.dot(jnp.cos(tb_ref[...]), w2_ref[2 * d_node:],
                     preferred_element_type=jnp.float32) + b2_ref[...])
    out_ref[...] = (jnp.dot(acc, w2_ref[0:d_node],
                            preferred_element_type=jnp.float32)
                    + jnp.dot(fs_ref[...], w2_ref[d_node:2 * d_node],
                              preferred_element_type=jnp.float32)
                    + cterm)


def _agg_final_body(fn_ref, on_ref, t_ref, n_ref, e_ref, ts_ref, fs_ref,
                    w1_ref, b1_ref, w2_ref, b2_ref, tw_ref, tb_ref,
                    out_ref, h1_ref, *, kk, d_node):
  j = pl.program_id(0)
  delta = ts_ref[...] - t_ref[0]
  tt = jnp.cos(delta * tw_ref[...] + tb_ref[...])
  e = e_ref[0]
  common = (jnp.dot(tt, w1_ref[d_node:2 * d_node],
                    preferred_element_type=jnp.float32)
            + jnp.dot(e, w1_ref[2 * d_node:],
                      preferred_element_type=jnp.float32)
            + b1_ref[...])
  m = (n_ref[0] != 0).astype(jnp.float32)
  a1 = jnp.maximum(jnp.dot(fn_ref[0], w1_ref[0:d_node],
                           preferred_element_type=jnp.float32) + common,
                   0.0) * m
  a2 = jnp.maximum(jnp.dot(on_ref[0], w1_ref[0:d_node],
                           preferred_element_type=jnp.float32) + common,
                   0.0) * m

  @pl.when(j == 0)
  def _():
    h1_ref[...] = a1
    out_ref[...] = a2

  @pl.when(j > 0)
  def _():
    h1_ref[...] += a1
    out_ref[...] += a2

  @pl.when(j == kk - 1)
  def _():
    cterm = (jnp.dot(jnp.cos(tb_ref[...]), w2_ref[2 * d_node:],
                     preferred_element_type=jnp.float32) + b2_ref[...])
    w2a = w2_ref[0:d_node]
    w2b = w2_ref[d_node:2 * d_node]
    src1 = (jnp.dot(h1_ref[...], w2a, preferred_element_type=jnp.float32)
            + jnp.dot(fs_ref[...], w2b, preferred_element_type=jnp.float32)
            + cterm)
    out_ref[...] = (jnp.dot(out_ref[...], w2a,
                            preferred_element_type=jnp.float32)
                    + jnp.dot(src1, w2b, preferred_element_type=jnp.float32)
                    + cterm)


def kernel(memory, source_nodes, timestamps, n_layers, n_neighbors,
           node_features, edge_features, neighbor_table, edge_idx_table,
           edge_time_table, time_w, time_b, W1, b1, W2, b2):
  del n_layers, n_neighbors
  n_nodes, d_node = node_features.shape
  n_edges, d_edge = edge_features.shape
  kk = neighbor_table.shape[1]
  bb = source_nodes.shape[0]
  m1 = bb * kk
  m2 = m1 * kk

  src = source_nodes.astype(jnp.int32)
  b1r = b1.reshape(1, d_node)
  b2r = b2.reshape(1, d_node)
  twr = time_w.reshape(1, d_node)
  tbr = time_b.reshape(1, d_node)

  ch_a = bb // NW
  gat_a = _make_sc_gather(bb, ch_a, True, False, kk, d_node, d_edge)
  n1, e1, t1, f_src = gat_a(neighbor_table, edge_idx_table, edge_time_table,
                            node_features, memory,
                            src.reshape(NW, ch_a))

  nbf = n1.T.reshape(-1)
  e1f = e1.T.reshape(-1)

  ch_b = 80
  nch_b = m1 // (NW * ch_b)
  gat_b = _make_sc_gather(m1, ch_b, True, True, kk, d_node, d_edge)
  n2, e2, t2, f_n1, ef1 = gat_b(neighbor_table, edge_idx_table,
                                edge_time_table, node_features, memory,
                                edge_features,
                                nbf.reshape(NW * nch_b, ch_b),
                                e1f.reshape(NW * nch_b, ch_b))

  n2f = n2.T.reshape(-1)
  e2f = e2.T.reshape(-1)

  ch_c = 128
  nch_c = m2 // (NW * ch_c)
  gat_c = _make_sc_gather(m2, ch_c, False, True, kk, d_node, d_edge)
  f_n2, ef2 = gat_c(node_features, memory, edge_features,
                    n2f.reshape(NW * nch_c, ch_c),
                    e2f.reshape(NW * nch_c, ch_c))

  bm = 1024
  g1 = m1 // bm
  tsf = jnp.tile(timestamps, kk).reshape(m1, 1)
  out1_nb = pl.pallas_call(
      functools.partial(_agg_big_body, kk=kk, d_node=d_node),
      grid=(g1, kk),
      in_specs=[
          pl.BlockSpec((1, bm, d_node), lambda g, j: (j, g, 0)),
          pl.BlockSpec((1, bm, 1), lambda g, j: (j, g, 0)),
          pl.BlockSpec((1, bm, 1), lambda g, j: (j, g, 0)),
          pl.BlockSpec((1, bm, d_edge), lambda g, j: (j, g, 0)),
          pl.BlockSpec((bm, 1), lambda g, j: (g, 0)),
          pl.BlockSpec((bm, d_node), lambda g, j: (g, 0)),
          pl.BlockSpec(W1.shape, lambda g, j: (0, 0)),
          pl.BlockSpec(b1r.shape, lambda g, j: (0, 0)),
          pl.BlockSpec(W2.shape, lambda g, j: (0, 0)),
          pl.BlockSpec(b2r.shape, lambda g, j: (0, 0)),
          pl.BlockSpec(twr.shape, lambda g, j: (0, 0)),
          pl.BlockSpec(tbr.shape, lambda g, j: (0, 0)),
      ],
      out_specs=pl.BlockSpec((bm, d_node), lambda g, j: (g, 0)),
      out_shape=jax.ShapeDtypeStruct((m1, d_node), jnp.float32),
  )(f_n2.reshape(kk, m1, d_node),
    t2.T.reshape(kk, m1, 1),
    n2.T.reshape(kk, m1, 1),
    ef2.reshape(kk, m1, d_edge),
    tsf, f_n1, W1, b1r, W2, b2r, twr, tbr)

  out = pl.pallas_call(
      functools.partial(_agg_final_body, kk=kk, d_node=d_node),
      grid=(kk,),
      in_specs=[
          pl.BlockSpec((1, bb, d_node), lambda j: (j, 0, 0)),
          pl.BlockSpec((1, bb, d_node), lambda j: (j, 0, 0)),
          pl.BlockSpec((1, bb, 1), lambda j: (j, 0, 0)),
          pl.BlockSpec((1, bb, 1), lambda j: (j, 0, 0)),
          pl.BlockSpec((1, bb, d_edge), lambda j: (j, 0, 0)),
          pl.BlockSpec((bb, 1), lambda j: (0, 0)),
          pl.BlockSpec((bb, d_node), lambda j: (0, 0)),
          pl.BlockSpec(W1.shape, lambda j: (0, 0)),
          pl.BlockSpec(b1r.shape, lambda j: (0, 0)),
          pl.BlockSpec(W2.shape, lambda j: (0, 0)),
          pl.BlockSpec(b2r.shape, lambda j: (0, 0)),
          pl.BlockSpec(twr.shape, lambda j: (0, 0)),
          pl.BlockSpec(tbr.shape, lambda j: (0, 0)),
      ],
      out_specs=pl.BlockSpec((bb, d_node), lambda j: (0, 0)),
      out_shape=jax.ShapeDtypeStruct((bb, d_node), jnp.float32),
      scratch_shapes=[pltpu.VMEM((bb, d_node), jnp.float32)],
  )(f_n1.reshape(kk, bb, d_node),
    out1_nb.reshape(kk, bb, d_node),
    t1.T.reshape(kk, bb, 1),
    n1.T.reshape(kk, bb, 1),
    ef1.reshape(kk, bb, d_edge),
    timestamps.reshape(bb, 1),
    f_src, W1, b1r, W2, b2r, twr, tbr)

  return out


# speedup vs baseline: 2.8274x; 2.8274x over previous
import functools

import jax
import jax.numpy as jnp
from jax import lax
from jax.experimental import pallas as pl
from jax.experimental.pallas import tpu as pltpu
from jax.experimental.pallas import tpu_sc as plsc

NC = 2
NS = 16
NW = NC * NS


def _make_sc_gather(n_idx, chunk, want_tables, want_edges, kk, d_node, d_edge):
  nch = n_idx // (NW * chunk)
  assert nch * NW * chunk == n_idx
  mesh = plsc.VectorSubcoreMesh(core_axis_name="c", subcore_axis_name="s")

  out_type = []
  if want_tables:
    out_type += [jax.ShapeDtypeStruct((n_idx, kk), jnp.int32),
                 jax.ShapeDtypeStruct((n_idx, kk), jnp.int32),
                 jax.ShapeDtypeStruct((n_idx, kk), jnp.float32)]
  out_type.append(jax.ShapeDtypeStruct((n_idx, d_node), jnp.float32))
  if want_edges:
    out_type.append(jax.ShapeDtypeStruct((n_idx, d_edge), jnp.float32))

  scratch = [pltpu.VMEM((nch, chunk), jnp.int32),
             pltpu.VMEM((chunk, d_node), jnp.float32),
             pltpu.SemaphoreType.DMA, pltpu.SemaphoreType.DMA]
  if want_tables:
    scratch += [pltpu.VMEM((chunk, kk), jnp.int32),
                pltpu.VMEM((chunk, kk), jnp.int32),
                pltpu.VMEM((chunk, kk), jnp.float32),
                pltpu.SemaphoreType.DMA]
  if want_edges:
    scratch += [pltpu.VMEM((nch, chunk), jnp.int32),
                pltpu.VMEM((chunk, d_edge), jnp.float32),
                pltpu.SemaphoreType.DMA]

  def body(*refs):
    it = iter(refs)
    if want_tables:
      nt_h, et_h, tt_h = next(it), next(it), next(it)
    nf_h, mem_h = next(it), next(it)
    if want_edges:
      ef_h = next(it)
    nidx_h = next(it)
    if want_edges:
      eidx_h = next(it)
    if want_tables:
      n_o, e_o, t_o = next(it), next(it), next(it)
    f_o = next(it)
    if want_edges:
      eo = next(it)
    nidx_v, bf = next(it), next(it)
    s0, s1 = next(it), next(it)
    if want_tables:
      bn, be, bt = next(it), next(it), next(it)
      s2 = next(it)
    if want_edges:
      eidx_v, bef = next(it), next(it)
      s5 = next(it)

    wid = lax.axis_index("s") * NC + lax.axis_index("c")

    pltpu.sync_copy(nidx_h.at[wid], nidx_v)
    if want_edges:
      pltpu.sync_copy(eidx_h.at[wid], eidx_v)

    for i in range(nch):
      base = (wid * nch + i) * chunk
      idx = nidx_v.at[i]
      cf = pltpu.async_copy(nf_h.at[idx], bf, s0)

      if want_tables:
        def tbody(g, carry):
          vec = nidx_v[i, pl.ds(g * 16, 16)]
          for l in range(16):
            row = g * 16 + l
            pltpu.async_copy(nt_h.at[pl.ds(vec[l], 1)], bn.at[pl.ds(row, 1)],
                             s2)
            pltpu.async_copy(et_h.at[pl.ds(vec[l], 1)], be.at[pl.ds(row, 1)],
                             s2)
            pltpu.async_copy(tt_h.at[pl.ds(vec[l], 1)], bt.at[pl.ds(row, 1)],
                             s2)
          return carry
        lax.fori_loop(0, chunk // 16, tbody, 0)
      if want_edges:
        def ebody(g, carry):
          vec = eidx_v[i, pl.ds(g * 16, 16)]
          for l in range(16):
            pltpu.async_copy(ef_h.at[pl.ds(vec[l], 1)],
                             bef.at[pl.ds(g * 16 + l, 1)], s5)
          return carry
        lax.fori_loop(0, chunk // 16, ebody, 0)

      cf.wait()
      cm = pltpu.async_copy(mem_h.at[idx], bf, s1, add=True)

      if want_tables:
        pltpu.make_async_copy(nt_h.at[pl.ds(0, chunk)], bn, s2).wait()
        pltpu.make_async_copy(et_h.at[pl.ds(0, chunk)], be, s2).wait()
        pltpu.make_async_copy(tt_h.at[pl.ds(0, chunk)], bt, s2).wait()
        pltpu.sync_copy(bn, n_o.at[pl.ds(base, chunk)])
        pltpu.sync_copy(be, e_o.at[pl.ds(base, chunk)])
        pltpu.sync_copy(bt, t_o.at[pl.ds(base, chunk)])
      if want_edges:
        pltpu.make_async_copy(ef_h.at[pl.ds(0, chunk)], bef, s5).wait()
        pltpu.sync_copy(bef, eo.at[pl.ds(base, chunk)])
      cm.wait()
      pltpu.sync_copy(bf, f_o.at[pl.ds(base, chunk)])

  return pl.kernel(body, out_type=tuple(out_type), mesh=mesh,
                   scratch_types=tuple(scratch))


def _agg_big_body(fn_ref, t_ref, n_ref, e_ref, ts_ref, fs_ref, w1_ref, b1_ref,
                  w2_ref, b2_ref, tw_ref, tb_ref, out_ref, *, kk, d_node):
  j = pl.program_id(1)
  delta = ts_ref[...] - t_ref[0]
  tt = jnp.cos(delta * tw_ref[...] + tb_ref[...])
  f = fn_ref[0]
  e = e_ref[0]
  pre = (jnp.dot(f, w1_ref[0:d_node], preferred_element_type=jnp.float32)
         + jnp.dot(tt, w1_ref[d_node:2 * d_node],
                   preferred_element_type=jnp.float32)
         + jnp.dot(e, w1_ref[2 * d_node:], preferred_element_type=jnp.float32)
         + b1_ref[...])
  m = (n_ref[0] != 0).astype(jnp.float32)
  h = jnp.maximum(pre, 0.0) * m

  @pl.when(j == 0)
  def _():
    out_ref[...] = h

  @pl.when(j > 0)
  def _():
    out_ref[...] += h

  @pl.when(j == kk - 1)
  def _():
    acc = out_ref[...]
    cterm = (jnp.dot(jnp.cos(tb_ref[...]), w2_ref[2 * d_node:],
                     preferred_element_type=jnp.float32) + b2_ref[...])
    out_ref[...] = (jnp.dot(acc, w2_ref[0:d_node],
                            preferred_element_type=jnp.float32)
                    + jnp.dot(fs_ref[...], w2_ref[d_node:2 * d_node],
                              preferred_element_type=jnp.float32)
                    + cterm)


def _agg_final_body(fn_ref, on_ref, t_ref, n_ref, e_ref, ts_ref, fs_ref,
                    w1_ref, b1_ref, w2_ref, b2_ref, tw_ref, tb_ref,
                    out_ref, h1_ref, *, kk, d_node):
  j = pl.program_id(0)
  delta = ts_ref[...] - t_ref[0]
  tt = jnp.cos(delta * tw_ref[...] + tb_ref[...])
  e = e_ref[0]
  common = (jnp.dot(tt, w1_ref[d_node:2 * d_node],
                    preferred_element_type=jnp.float32)
            + jnp.dot(e, w1_ref[2 * d_node:],
                      preferred_element_type=jnp.float32)
            + b1_ref[...])
  m = (n_ref[0] != 0).astype(jnp.float32)
  a1 = jnp.maximum(jnp.dot(fn_ref[0], w1_ref[0:d_node],
                           preferred_element_type=jnp.float32) + common,
                   0.0) * m
  a2 = jnp.maximum(jnp.dot(on_ref[0], w1_ref[0:d_node],
                           preferred_element_type=jnp.float32) + common,
                   0.0) * m

  @pl.when(j == 0)
  def _():
    h1_ref[...] = a1
    out_ref[...] = a2

  @pl.when(j > 0)
  def _():
    h1_ref[...] += a1
    out_ref[...] += a2

  @pl.when(j == kk - 1)
  def _():
    cterm = (jnp.dot(jnp.cos(tb_ref[...]), w2_ref[2 * d_node:],
                     preferred_element_type=jnp.float32) + b2_ref[...])
    w2a = w2_ref[0:d_node]
    w2b = w2_ref[d_node:2 * d_node]
    src1 = (jnp.dot(h1_ref[...], w2a, preferred_element_type=jnp.float32)
            + jnp.dot(fs_ref[...], w2b, preferred_element_type=jnp.float32)
            + cterm)
    out_ref[...] = (jnp.dot(out_ref[...], w2a,
                            preferred_element_type=jnp.float32)
                    + jnp.dot(src1, w2b, preferred_element_type=jnp.float32)
                    + cterm)


def kernel(memory, source_nodes, timestamps, n_layers, n_neighbors,
           node_features, edge_features, neighbor_table, edge_idx_table,
           edge_time_table, time_w, time_b, W1, b1, W2, b2):
  del n_layers, n_neighbors
  n_nodes, d_node = node_features.shape
  n_edges, d_edge = edge_features.shape
  kk = neighbor_table.shape[1]
  bb = source_nodes.shape[0]
  m1 = bb * kk
  m2 = m1 * kk

  src = source_nodes.astype(jnp.int32)
  b1r = b1.reshape(1, d_node)
  b2r = b2.reshape(1, d_node)
  twr = time_w.reshape(1, d_node)
  tbr = time_b.reshape(1, d_node)

  ch_a = bb // NW
  gat_a = _make_sc_gather(bb, ch_a, True, False, kk, d_node, d_edge)
  n1, e1, t1, f_src = gat_a(neighbor_table, edge_idx_table, edge_time_table,
                            node_features, memory,
                            src.reshape(NW, 1, ch_a))

  nbf = n1.T.reshape(-1)
  e1f = e1.T.reshape(-1)

  ch_b = 80
  nch_b = m1 // (NW * ch_b)
  gat_b = _make_sc_gather(m1, ch_b, True, True, kk, d_node, d_edge)
  n2, e2, t2, f_n1, ef1 = gat_b(neighbor_table, edge_idx_table,
                                edge_time_table, node_features, memory,
                                edge_features,
                                nbf.reshape(NW, nch_b, ch_b),
                                e1f.reshape(NW, nch_b, ch_b))

  n2f = n2.T.reshape(-1)
  e2f = e2.T.reshape(-1)

  ch_c = 128
  nch_c = m2 // (NW * ch_c)
  gat_c = _make_sc_gather(m2, ch_c, False, True, kk, d_node, d_edge)
  f_n2, ef2 = gat_c(node_features, memory, edge_features,
                    n2f.reshape(NW, nch_c, ch_c),
                    e2f.reshape(NW, nch_c, ch_c))

  bm = 1024
  g1 = m1 // bm
  tsf = jnp.tile(timestamps, kk).reshape(m1, 1)
  out1_nb = pl.pallas_call(
      functools.partial(_agg_big_body, kk=kk, d_node=d_node),
      grid=(g1, kk),
      in_specs=[
          pl.BlockSpec((1, bm, d_node), lambda g, j: (j, g, 0)),
          pl.BlockSpec((1, bm, 1), lambda g, j: (j, g, 0)),
          pl.BlockSpec((1, bm, 1), lambda g, j: (j, g, 0)),
          pl.BlockSpec((1, bm, d_edge), lambda g, j: (j, g, 0)),
          pl.BlockSpec((bm, 1), lambda g, j: (g, 0)),
          pl.BlockSpec((bm, d_node), lambda g, j: (g, 0)),
          pl.BlockSpec(W1.shape, lambda g, j: (0, 0)),
          pl.BlockSpec(b1r.shape, lambda g, j: (0, 0)),
          pl.BlockSpec(W2.shape, lambda g, j: (0, 0)),
          pl.BlockSpec(b2r.shape, lambda g, j: (0, 0)),
          pl.BlockSpec(twr.shape, lambda g, j: (0, 0)),
          pl.BlockSpec(tbr.shape, lambda g, j: (0, 0)),
      ],
      out_specs=pl.BlockSpec((bm, d_node), lambda g, j: (g, 0)),
      out_shape=jax.ShapeDtypeStruct((m1, d_node), jnp.float32),
  )(f_n2.reshape(kk, m1, d_node),
    t2.T.reshape(kk, m1, 1),
    n2.T.reshape(kk, m1, 1),
    ef2.reshape(kk, m1, d_edge),
    tsf, f_n1, W1, b1r, W2, b2r, twr, tbr)

  out = pl.pallas_call(
      functools.partial(_agg_final_body, kk=kk, d_node=d_node),
      grid=(kk,),
      in_specs=[
          pl.BlockSpec((1, bb, d_node), lambda j: (j, 0, 0)),
          pl.BlockSpec((1, bb, d_node), lambda j: (j, 0, 0)),
          pl.BlockSpec((1, bb, 1), lambda j: (j, 0, 0)),
          pl.BlockSpec((1, bb, 1), lambda j: (j, 0, 0)),
          pl.BlockSpec((1, bb, d_edge), lambda j: (j, 0, 0)),
          pl.BlockSpec((bb, 1), lambda j: (0, 0)),
          pl.BlockSpec((bb, d_node), lambda j: (0, 0)),
          pl.BlockSpec(W1.shape, lambda j: (0, 0)),
          pl.BlockSpec(b1r.shape, lambda j: (0, 0)),
          pl.BlockSpec(W2.shape, lambda j: (0, 0)),
          pl.BlockSpec(b2r.shape, lambda j: (0, 0)),
          pl.BlockSpec(twr.shape, lambda j: (0, 0)),
          pl.BlockSpec(tbr.shape, lambda j: (0, 0)),
      ],
      out_specs=pl.BlockSpec((bb, d_node), lambda j: (0, 0)),
      out_shape=jax.ShapeDtypeStruct((bb, d_node), jnp.float32),
      scratch_shapes=[pltpu.VMEM((bb, d_node), jnp.float32)],
  )(f_n1.reshape(kk, bb, d_node),
    out1_nb.reshape(kk, bb, d_node),
    t1.T.reshape(kk, bb, 1),
    n1.T.reshape(kk, bb, 1),
    ef1.reshape(kk, bb, d_edge),
    timestamps.reshape(bb, 1),
    f_src, W1, b1r, W2, b2r, twr, tbr)

  return out
